# Initial kernel scaffold; baseline (speedup 1.0000x reference)
import functools

import jax
import jax.numpy as jnp
from jax import lax
from jax.experimental import pallas as pl
from jax.experimental.pallas import tpu as pltpu
from jax.experimental.pallas import tpu_sc as plsc

N = 10000
D = 128
E = 320000

NC = 2
NS = 16
NW = NC * NS
E_PER_TILE = E // NW
CHUNK = 80
NCHUNK = E_PER_TILE // CHUNK
ROWS_PER_TILE = N // NS

_mesh = plsc.VectorSubcoreMesh(core_axis_name="c", subcore_axis_name="s")


def _fill_rows(buf, rows, width, value):
    vec = jnp.full((16,), value, jnp.float32)

    def body(i, carry):
        for j in range(width // 16):
            buf[i, pl.ds(j * 16, 16)] = vec
        return carry

    lax.fori_loop(0, rows, body, 0)


@functools.partial(
    pl.kernel,
    out_type=jax.ShapeDtypeStruct((NC, N, 16), jnp.float32),
    mesh=_mesh,
    scratch_types=[
        pltpu.VMEM((NCHUNK, CHUNK), jnp.int32),
        pltpu.VMEM((125, 16), jnp.float32),
        pltpu.VMEM_SHARED((N, 16), jnp.float32),
    ],
)
def _deg_kernel(dst_hbm, out_hbm, dst_v, buf_v, deg_sh):
    c = lax.axis_index("c")
    s = lax.axis_index("s")
    wid = s * NC + c
    pltpu.sync_copy(dst_hbm.at[wid], dst_v)
    _fill_rows(buf_v, 125, 16, 0.0)
    for r in range(ROWS_PER_TILE // 125):
        pltpu.sync_copy(buf_v, deg_sh.at[pl.ds(s * ROWS_PER_TILE + r * 125, 125)])
    _fill_rows(buf_v, CHUNK, 16, 1.0)
    plsc.subcore_barrier()

    def step(k, carry):
        pltpu.sync_copy(buf_v.at[pl.ds(0, CHUNK)], deg_sh.at[dst_v.at[k]], add=True)
        return carry

    lax.fori_loop(0, NCHUNK, step, 0)
    plsc.subcore_barrier()
    pltpu.sync_copy(
        deg_sh.at[pl.ds(s * ROWS_PER_TILE, ROWS_PER_TILE)],
        out_hbm.at[c, pl.ds(s * ROWS_PER_TILE, ROWS_PER_TILE)],
    )


@functools.partial(
    pl.kernel,
    out_type=jax.ShapeDtypeStruct((NC, N, D), jnp.float32),
    mesh=_mesh,
    scratch_types=[
        pltpu.VMEM((NCHUNK, CHUNK), jnp.int32),
        pltpu.VMEM((NCHUNK, CHUNK), jnp.int32),
        pltpu.VMEM((CHUNK, D), jnp.float32),
        pltpu.VMEM((125, D), jnp.float32),
        pltpu.VMEM_SHARED((N, D), jnp.float32),
        pltpu.SemaphoreType.DMA,
    ],
)
def _msg_kernel(z_hbm, src_hbm, dst_hbm, out_hbm, src_v, dst_v, gbuf, zbuf,
                agg_sh, sem):
    c = lax.axis_index("c")
    s = lax.axis_index("s")
    wid = s * NC + c
    pltpu.sync_copy(src_hbm.at[wid], src_v)
    pltpu.sync_copy(dst_hbm.at[wid], dst_v)
    _fill_rows(zbuf, 125, D, 0.0)
    for r in range(ROWS_PER_TILE // 125):
        pltpu.sync_copy(zbuf, agg_sh.at[pl.ds(s * ROWS_PER_TILE + r * 125, 125)])
    plsc.subcore_barrier()

    def step(k, carry):
        pltpu.async_copy(z_hbm.at[src_v.at[k]], gbuf, sem).wait()
        pltpu.sync_copy(gbuf, agg_sh.at[dst_v.at[k]], add=True)
        return carry

    lax.fori_loop(0, NCHUNK, step, 0)
    plsc.subcore_barrier()
    pltpu.sync_copy(
        agg_sh.at[pl.ds(s * ROWS_PER_TILE, ROWS_PER_TILE)],
        out_hbm.at[c, pl.ds(s * ROWS_PER_TILE, ROWS_PER_TILE)],
    )


def _dinv_from_parts(dp_ref):
    dp0 = dp_ref[0, :, 0:1]
    dp1 = dp_ref[1, :, 0:1]
    return lax.rsqrt(dp0 + dp1 + 1.0)


def _tc1_body(x_ref, w_ref, dp_ref, z_ref):
    dinv = _dinv_from_parts(dp_ref)
    z_ref[...] = dinv * jnp.dot(x_ref[...], w_ref[...],
                                preferred_element_type=jnp.float32)


def _tc2_body(agg_ref, z1_ref, dp_ref, b1_ref, w2_ref, z2_ref):
    dinv = _dinv_from_parts(dp_ref)
    pre = dinv * (agg_ref[0] + agg_ref[1] + z1_ref[...]) + b1_ref[...]
    h = jnp.maximum(pre, 0.0)
    z2_ref[...] = dinv * jnp.dot(h, w2_ref[...],
                                 preferred_element_type=jnp.float32)


def _tc3_body(agg_ref, z2_ref, dp_ref, b2_ref, out_ref):
    dinv = _dinv_from_parts(dp_ref)
    out_ref[...] = dinv * (agg_ref[0] + agg_ref[1] + z2_ref[...]) + b2_ref[...]


_tc1 = pl.pallas_call(
    _tc1_body, out_shape=jax.ShapeDtypeStruct((N, D), jnp.float32))
_tc2 = pl.pallas_call(
    _tc2_body, out_shape=jax.ShapeDtypeStruct((N, D), jnp.float32))
_tc3 = pl.pallas_call(
    _tc3_body, out_shape=jax.ShapeDtypeStruct((N, D), jnp.float32))


def kernel(x, edge_index, W1, b1, W2, b2):
    ei = edge_index.astype(jnp.int32)
    src = ei[0].reshape(NW, NCHUNK, CHUNK)
    dst = ei[1].reshape(NW, NCHUNK, CHUNK)
    b1r = b1.reshape(1, D)
    b2r = b2.reshape(1, D)

    deg_part = _deg_kernel(dst)
    z1 = _tc1(x, W1, deg_part)
    agg1 = _msg_kernel(z1, src, dst)
    z2 = _tc2(agg1, z1, deg_part, b1r, W2)
    agg2 = _msg_kernel(z2, src, dst)
    out = _tc3(agg2, z2, deg_part, b2r)
    return out


# speedup vs baseline: 18.0594x; 18.0594x over previous
import functools

import jax
import jax.numpy as jnp
from jax import lax
from jax.experimental import pallas as pl
from jax.experimental.pallas import tpu as pltpu
from jax.experimental.pallas import tpu_sc as plsc

N = 10000
NP = 10240
D = 128
E = 320000

NC = 2
NS = 16
NW = NC * NS
CHUNK = 80
E_PER_TILE = E // NW
NCHUNK = E_PER_TILE // CHUNK
BLK = 25
NBLK = NCHUNK // BLK
ROWS_PER_TILE = NP // NS

_mesh = plsc.VectorSubcoreMesh(
    core_axis_name="c", subcore_axis_name="s", num_cores=NC, num_subcores=NS)


def _fill_rows(buf, rows, width, value):
    vec = jnp.full((16,), value, jnp.float32)

    def body(i, carry):
        for j in range(width // 16):
            buf[i, pl.ds(j * 16, 16)] = vec
        return carry

    lax.fori_loop(0, rows, body, 0)


def _deg_body(dst_hbm, out_hbm, dst_v, buf_v, deg_sh):
    c = lax.axis_index("c")
    s = lax.axis_index("s")
    wid = s * NC + c
    _fill_rows(buf_v, CHUNK, D, 0.0)
    for r in range(ROWS_PER_TILE // CHUNK):
        pltpu.sync_copy(buf_v, deg_sh.at[pl.ds(s * ROWS_PER_TILE + r * CHUNK, CHUNK)])
    _fill_rows(buf_v, CHUNK, D, 1.0)
    plsc.subcore_barrier()

    def blk_body(b, carry):
        pltpu.sync_copy(dst_hbm.at[wid, b], dst_v)

        def step(k, carry2):
            pltpu.sync_copy(buf_v, deg_sh.at[dst_v.at[k]], add=True)
            return carry2

        return lax.fori_loop(0, BLK, step, carry)

    lax.fori_loop(0, NBLK, blk_body, 0)
    plsc.subcore_barrier()
    pltpu.sync_copy(
        deg_sh.at[pl.ds(s * ROWS_PER_TILE, ROWS_PER_TILE)],
        out_hbm.at[c, pl.ds(s * ROWS_PER_TILE, ROWS_PER_TILE)],
    )


def _msg_body(z_hbm, src_hbm, dst_hbm, out_hbm, src_v, dst_v, gbuf,
              agg_sh, sem):
    c = lax.axis_index("c")
    s = lax.axis_index("s")
    wid = s * NC + c
    _fill_rows(gbuf, CHUNK, D, 0.0)
    for r in range(ROWS_PER_TILE // CHUNK):
        pltpu.sync_copy(gbuf, agg_sh.at[pl.ds(s * ROWS_PER_TILE + r * CHUNK, CHUNK)])
    plsc.subcore_barrier()

    def blk_body(b, carry):
        pltpu.sync_copy(src_hbm.at[wid, b], src_v)
        pltpu.sync_copy(dst_hbm.at[wid, b], dst_v)

        def step(k, carry2):
            pltpu.async_copy(z_hbm.at[src_v.at[k]], gbuf, sem).wait()
            pltpu.sync_copy(gbuf, agg_sh.at[dst_v.at[k]], add=True)
            return carry2

        return lax.fori_loop(0, BLK, step, carry)

    lax.fori_loop(0, NBLK, blk_body, 0)
    plsc.subcore_barrier()
    pltpu.sync_copy(
        agg_sh.at[pl.ds(s * ROWS_PER_TILE, ROWS_PER_TILE)],
        out_hbm.at[c, pl.ds(s * ROWS_PER_TILE, ROWS_PER_TILE)],
    )


def _make_deg_kernel(interpret=False):
    return pl.kernel(
        _deg_body,
        out_type=jax.ShapeDtypeStruct((NC, NP, D), jnp.float32),
        mesh=_mesh,
        scratch_types=[
            pltpu.VMEM((BLK, CHUNK), jnp.int32),
            pltpu.VMEM((CHUNK, D), jnp.float32),
            pltpu.VMEM_SHARED((NP, D), jnp.float32),
        ],
        interpret=interpret,
    )


def _make_msg_kernel(interpret=False):
    return pl.kernel(
        _msg_body,
        out_type=jax.ShapeDtypeStruct((NC, NP, D), jnp.float32),
        mesh=_mesh,
        scratch_types=[
            pltpu.VMEM((BLK, CHUNK), jnp.int32),
            pltpu.VMEM((BLK, CHUNK), jnp.int32),
            pltpu.VMEM((CHUNK, D), jnp.float32),
            pltpu.VMEM_SHARED((NP, D), jnp.float32),
            pltpu.SemaphoreType.DMA,
        ],
        interpret=interpret,
    )


_deg_kernel = _make_deg_kernel()
_msg_kernel = _make_msg_kernel()


def _dinv_from_parts(dp_ref):
    dp0 = dp_ref[0, 0:N, 0:1]
    dp1 = dp_ref[1, 0:N, 0:1]
    return lax.rsqrt(dp0 + dp1 + 1.0)


def _cat_agg(agg_ref):
    return agg_ref[0, 0:N, :] + agg_ref[1, 0:N, :]


def _tc1_body(x_ref, w_ref, dp_ref, z_ref):
    dinv = _dinv_from_parts(dp_ref)
    z_ref[...] = dinv * jnp.dot(x_ref[...], w_ref[...],
                                preferred_element_type=jnp.float32)


def _tc2_body(agg_ref, z1_ref, dp_ref, b1_ref, w2_ref, z2_ref):
    dinv = _dinv_from_parts(dp_ref)
    pre = dinv * (_cat_agg(agg_ref) + z1_ref[...]) + b1_ref[...]
    h = jnp.maximum(pre, 0.0)
    z2_ref[...] = dinv * jnp.dot(h, w2_ref[...],
                                 preferred_element_type=jnp.float32)


def _tc3_body(agg_ref, z2_ref, dp_ref, b2_ref, out_ref):
    dinv = _dinv_from_parts(dp_ref)
    out_ref[...] = dinv * (_cat_agg(agg_ref) + z2_ref[...]) + b2_ref[...]


_tc1 = pl.pallas_call(
    _tc1_body, out_shape=jax.ShapeDtypeStruct((N, D), jnp.float32))
_tc2 = pl.pallas_call(
    _tc2_body, out_shape=jax.ShapeDtypeStruct((N, D), jnp.float32))
_tc3 = pl.pallas_call(
    _tc3_body, out_shape=jax.ShapeDtypeStruct((N, D), jnp.float32))


def kernel(x, edge_index, W1, b1, W2, b2):
    ei = edge_index.astype(jnp.int32)
    src = ei[0].reshape(NW, NBLK, BLK, CHUNK)
    dst = ei[1].reshape(NW, NBLK, BLK, CHUNK)
    b1r = b1.reshape(1, D)
    b2r = b2.reshape(1, D)

    deg_part = _deg_kernel(dst)
    z1 = _tc1(x, W1, deg_part)
    agg1 = _msg_kernel(z1, src, dst)
    z2 = _tc2(agg1, z1, deg_part, b1r, W2)
    agg2 = _msg_kernel(z2, src, dst)
    out = _tc3(agg2, z2, deg_part, b2r)
    return out
